# baseline (speedup 1.0000x reference)
import functools

import jax
import jax.numpy as jnp
import numpy as np
from jax import lax
from jax.experimental import pallas as pl
from jax.experimental.pallas import tpu as pltpu
from jax.experimental.pallas import tpu_sc as plsc

F32 = jnp.float32


def _cayley_np():
    blades = [(), (1,), (2,), (3,), (1, 2), (1, 3), (2, 3), (1, 2, 3)]
    index = {b: i for i, b in enumerate(blades)}
    C = np.zeros((8, 8, 8), dtype=np.float32)
    for i, a in enumerate(blades):
        for j, b in enumerate(blades):
            lst = list(a) + list(b)
            sign = 1.0
            changed = True
            while changed:
                changed = False
                for k in range(len(lst) - 1):
                    if lst[k] > lst[k + 1]:
                        lst[k], lst[k + 1] = lst[k + 1], lst[k]
                        sign = -sign
                        changed = True
            out = []
            k = 0
            while k < len(lst):
                if k + 1 < len(lst) and lst[k] == lst[k + 1]:
                    k += 2
                else:
                    out.append(lst[k])
                    k += 1
            C[i, j, index[tuple(out)]] += sign
    return C


_C = _cayley_np()
_I16 = np.eye(16, dtype=np.float32)
_TT = np.kron(np.ones((1, 8), np.float32), _I16)
_RS16 = np.kron(np.ones((8, 1), np.float32), _I16)
_RS32 = np.kron(np.ones((8, 1), np.float32), np.eye(32, dtype=np.float32))
_PERM = np.stack([np.kron(_C[:, j, :], _I16) for j in range(8)])


BF16 = jnp.bfloat16


def _mm(x, w):
    return jnp.dot(x.astype(BF16), w, preferred_element_type=F32)


def _kron8(Wt):
    return jnp.kron(jnp.eye(8, dtype=F32), Wt)


def _b0(b, h):
    return jnp.concatenate([b, jnp.zeros((7 * h,), F32)]).reshape(1, -1)


def _row(x):
    return x.reshape(1, -1)


def _prep(lin, gp, ln, has_dpos):
    W = {}
    dlT, drT, whT = lin['dl_W'].T, lin['dr_W'].T, lin['wh_W'].T
    W['VA'] = jnp.concatenate(
        [_kron8(dlT[:16]), _kron8(drT[:16]), _kron8(whT[:16])], axis=1)
    W['VB'] = jnp.concatenate(
        [_kron8(dlT[16:]), _kron8(drT[16:]), _kron8(whT[16:])], axis=1)
    W['bV'] = jnp.concatenate(
        [_b0(lin['dl_b'], 32), _b0(lin['dr_b'], 32), _b0(lin['wh_b'], 32)],
        axis=1)
    wsT = lin['ws_W'].T
    W['wsA'], W['wsB'] = wsT[0:64], wsT[64:128]
    if has_dpos:
        W['wsD'] = wsT[128:129]
        wsV = wsT[129:]
    else:
        wsV = wsT[128:]
    W['dws'] = lin['do_W'].T @ wsV
    W['ws_b'] = _row(lin['ws_b'])
    W['Kwv'] = _kron8(lin['wv_W'].T)
    W['bwv'] = _b0(lin['wv_b'], 16)
    W['wsvT'] = lin['wsv_W'].T
    W['bwsv'] = _row(lin['wsv_b'])
    Kl = _kron8(gp['gp_l_W'].T)
    blv = jnp.concatenate([gp['gp_l_b'], jnp.zeros((112,), F32)])
    P = jnp.asarray(_PERM)
    L = jnp.einsum('ab,jbc->jac', Kl, P)
    bL = jnp.einsum('b,jbc->jc', blv, P)
    for j in range(8):
        W['L' + str(j)] = L[j]
        W['bL' + str(j)] = bL[j].reshape(1, 128)
    W['gKr'] = _kron8(gp['gp_r_W'].T)
    W['gbr'] = _b0(gp['gp_r_b'], 16)
    W['Tbig'] = jnp.asarray(np.kron(np.eye(8, dtype=np.float32), _TT))
    W['Ko'] = _kron8(gp['gp_o_W'].T)
    W['bo'] = _b0(gp['gp_o_b'], 16)
    W['a1'] = _row(jnp.tile(gp['gp_ln_a'], 8))
    W['Kw2'] = _kron8(gp['wv_W'].T)
    W['bw2'] = _b0(gp['wv_b'], 16)
    s2T = gp['s2v_W'].T
    W['S1'], W['S2'] = s2T[:16], s2T[16:]
    W['s2b'] = _row(gp['s2v_b'])
    W['E0'] = jnp.asarray(
        np.concatenate([_I16, np.zeros((16, 112), np.float32)], axis=1))
    W['mask0'] = jnp.asarray(
        np.concatenate([np.zeros((1, 16)), np.ones((1, 112))],
                       axis=1).astype(np.float32))
    W['g'] = _row(ln['ln_g'])
    W['b'] = _row(ln['ln_b'])
    W['a2'] = _row(jnp.tile(ln['mv_a'], 8))
    W['RS16'] = jnp.asarray(_RS16)
    W['RS32'] = jnp.asarray(_RS32)
    W['TT'] = jnp.asarray(_TT)
    for k in (('VA', 'VB', 'wsA', 'wsB', 'dws', 'Kwv', 'wsvT', 'gKr',
               'Tbig', 'Ko', 'Kw2', 'S1', 'S2', 'E0', 'RS16', 'RS32', 'TT')
              + tuple('L' + str(j) for j in range(8))):
        W[k] = W[k].astype(BF16)
    return W


def _cvp_chain(sA, sB, dpos, vA, vB, W):
    rs16, rs32, tt = W['RS16'], W['RS32'], W['TT']
    y = _mm(vA, W['VA']) + _mm(vB, W['VB']) + W['bV']
    vl, vr, vh = y[:, 0:256], y[:, 256:512], y[:, 512:768]
    dot = _mm(vl * vr, rs32)
    s_out = _mm(sA, W['wsA']) + _mm(sB, W['wsB']) + _mm(dot, W['dws']) + W['ws_b']
    if dpos is not None:
        s_out = s_out + dpos * W['wsD']
    v_out = _mm(vh, W['Kwv']) + W['bwv']
    gate = _mm(jax.nn.sigmoid(s_out), W['wsvT']) + W['bwsv']
    v_out = v_out * _mm(jax.nn.sigmoid(gate), tt)
    ms = jnp.maximum(s_out, 0.0)
    mv = v_out
    vr2 = _mm(mv, W['gKr']) + W['gbr']
    vrt = _mm(vr2, W['Tbig'])
    vo = (_mm(mv, W['L0']) + W['bL0']) * vrt[:, 0:128]
    for j in range(1, 8):
        vo = vo + (_mm(mv, W['L' + str(j)]) + W['bL' + str(j)]) \
            * vrt[:, 128 * j:128 * (j + 1)]
    vo = _mm(vo, W['Ko']) + W['bo']
    nrm = jnp.sqrt(_mm(vo * vo, rs16))
    vo = W['a1'] * vo / (jnp.mean(nrm, axis=1, keepdims=True) + 1e-6)
    v2 = vo + mv
    sc0 = v2[:, 0:16]
    g = jax.nn.sigmoid(np.float32(np.sqrt(2.0 / np.pi)) *
                       (2.0 * (sc0 + 0.044715 * sc0 ** 3)))
    v2 = v2 * _mm(g, tt)
    v2 = _mm(v2, W['Kw2']) + W['bw2']
    scal = _mm(v2[:, 0:16], W['S1']) + _mm(ms, W['S2']) + W['s2b']
    v2 = v2 * W['mask0'] + _mm(scal, W['E0'])
    mu = jnp.mean(ms, axis=1, keepdims=True)
    var = jnp.mean((ms - mu) ** 2, axis=1, keepdims=True)
    ms = (ms - mu) * lax.rsqrt(var + 1e-5) * W['g'] + W['b']
    nb = jnp.sqrt(_mm(v2 * v2, rs16))
    v2 = W['a2'] * v2 / (jnp.mean(nb, axis=1, keepdims=True) + 1e-6)
    return ms, v2


def _edge_body(rec_ref, send_ref, *rest, names, halves):
    outs_ref, outv_ref = rest[-2], rest[-1]
    W = dict(zip(names, [r[...] for r in rest[:-2]]))
    hb = rec_ref.shape[0] // halves
    for h in range(halves):
        sl = slice(h * hb, (h + 1) * hb)
        R = rec_ref[sl, :]
        S = send_ref[sl, :]
        dpos = S[:, 64:65] - R[:, 64:65]
        ms, mv = _cvp_chain(R[:, 0:64], S[:, 0:64], dpos,
                            R[:, 128:256], S[:, 128:256], W)
        outs_ref[sl, :] = jnp.concatenate([ms, jnp.zeros_like(ms)], axis=1)
        outv_ref[sl, :] = mv


def _node_body(s_ref, v_ref, as0_ref, as1_ref, av0_ref, av1_ref, *rest, names):
    os_ref, ov_ref = rest[-2], rest[-1]
    W = dict(zip(names, [r[...] for r in rest[:-2]]))
    aggs = as0_ref[...] + as1_ref[...]
    aggv = av0_ref[...] + av1_ref[...]
    ms, mv = _cvp_chain(s_ref[...], aggs[:, 0:64], None,
                        v_ref[...], aggv, W)
    os_ref[...] = ms
    ov_ref[...] = mv


def _pick_block(n, prefs):
    for b in prefs:
        if n % b == 0:
            return b
    return 1


def _full_spec(w):
    nd = w.ndim
    return pl.BlockSpec(w.shape, lambda i, _nd=nd: (0,) * _nd)


def _gather_edges(table, rec, send):
    N, D = table.shape
    E = rec.shape[0]
    NW = 32
    epw = E // NW
    CH = _pick_block(epw, (400, 320, 200, 160, 80, 40, 8))
    iters = epw // CH
    mesh = plsc.VectorSubcoreMesh(core_axis_name="c", subcore_axis_name="s")

    @functools.partial(
        pl.kernel, mesh=mesh,
        out_type=[jax.ShapeDtypeStruct((E, D), F32),
                  jax.ShapeDtypeStruct((E, D), F32)],
        scratch_types=[pltpu.VMEM((CH,), jnp.int32),
                       pltpu.VMEM((CH, D), F32),
                       pltpu.SemaphoreType.DMA],
    )
    def k(table_h, rec_h, send_h, rec_o, send_o, idx_v, rows_v, sem):
        wid = lax.axis_index("s") * 2 + lax.axis_index("c")
        base = wid * epw

        def body(t, carry):
            off = base + t * CH
            pltpu.sync_copy(rec_h.at[pl.ds(off, CH)], idx_v)
            pltpu.async_copy(table_h.at[idx_v], rows_v, sem).wait()
            pltpu.sync_copy(rows_v, rec_o.at[pl.ds(off, CH)])
            pltpu.sync_copy(send_h.at[pl.ds(off, CH)], idx_v)
            pltpu.async_copy(table_h.at[idx_v], rows_v, sem).wait()
            pltpu.sync_copy(rows_v, send_o.at[pl.ds(off, CH)])
            return carry

        lax.fori_loop(0, iters, body, 0)

    return k(table, rec, send)


def _scatter_msgs(msg, rec, zeros_nd):
    E, Dm = msg.shape
    N = zeros_nd.shape[0]
    NW = 32
    epw = E // NW
    CH = _pick_block(epw, (200, 160, 80, 40, 8))
    iters = epw // CH
    RC = _pick_block(N, (200, 400, 80, 40, 16, 8))
    n_chunks = N // RC
    per_tile = -(-n_chunks // 16)
    mesh = plsc.VectorSubcoreMesh(core_axis_name="c", subcore_axis_name="s")

    @functools.partial(
        pl.kernel, mesh=mesh,
        out_type=jax.ShapeDtypeStruct((2, N, Dm), F32),
        scratch_types=[pltpu.VMEM((CH,), jnp.int32),
                       pltpu.VMEM((CH, Dm), F32),
                       pltpu.VMEM_SHARED((N, Dm), F32)],
    )
    def k(msg_h, rec_h, zero_h, out_h, idx_v, rows_v, acc):
        c = lax.axis_index("c")
        s = lax.axis_index("s")
        wid = s * 2 + c

        def rows_body(t, carry, copy_out):
            cid = s * per_tile + t

            @pl.when(cid < n_chunks)
            def _():
                off = cid * RC
                if copy_out:
                    pltpu.sync_copy(acc.at[pl.ds(off, RC)],
                                    out_h.at[c, pl.ds(off, RC)])
                else:
                    pltpu.sync_copy(zero_h.at[pl.ds(off, RC)],
                                    acc.at[pl.ds(off, RC)])
            return carry

        lax.fori_loop(0, per_tile, functools.partial(rows_body, copy_out=False), 0)
        plsc.subcore_barrier()

        def body(t, carry):
            off = wid * epw + t * CH
            pltpu.sync_copy(rec_h.at[pl.ds(off, CH)], idx_v)
            pltpu.sync_copy(msg_h.at[pl.ds(off, CH)], rows_v)
            pltpu.sync_copy(rows_v, acc.at[idx_v], add=True)
            return carry

        lax.fori_loop(0, iters, body, 0)
        plsc.subcore_barrier()
        lax.fori_loop(0, per_tile, functools.partial(rows_body, copy_out=True), 0)

    return k(msg, rec, zeros_nd)


def kernel(s, v, pos, edge_index, params):
    N = s.shape[0]
    E = edge_index.shape[1]
    vflat = v.transpose(0, 2, 1).reshape(N, 128)
    table = jnp.concatenate(
        [s, pos[:, None], jnp.zeros((N, 63), F32), vflat], axis=1)
    send = edge_index[0]
    rec = edge_index[1]

    rec_feat, send_feat = _gather_edges(table, rec, send)

    We = _prep(params['edge_lin'], params['edge_gp'], params['edge_ln'], True)
    enames = tuple(We.keys())
    evals = [We[k] for k in enames]
    BE = _pick_block(E, (1024, 512, 640, 400, 320, 256, 128, 64, 32, 8))
    msg_s, msg_v = pl.pallas_call(
        functools.partial(_edge_body, names=enames, halves=2 if BE % 16 == 0 else 1),
        grid=(E // BE,),
        in_specs=[pl.BlockSpec((BE, 256), lambda i: (i, 0)),
                  pl.BlockSpec((BE, 256), lambda i: (i, 0))] +
                 [_full_spec(w) for w in evals],
        out_specs=[pl.BlockSpec((BE, 128), lambda i: (i, 0)),
                   pl.BlockSpec((BE, 128), lambda i: (i, 0))],
        out_shape=[jax.ShapeDtypeStruct((E, 128), F32),
                   jax.ShapeDtypeStruct((E, 128), F32)],
    )(rec_feat, send_feat, *evals)

    zeros_nd = jnp.zeros((N, 128), F32)
    agg_s = _scatter_msgs(msg_s, rec, zeros_nd)
    agg_v = _scatter_msgs(msg_v, rec, zeros_nd)

    Wn = _prep(params['node_lin'], params['node_gp'], params['node_ln'], False)
    nnames = tuple(Wn.keys())
    nvals = [Wn[k] for k in nnames]
    BN = _pick_block(N, (1000, 500, 250, 200, 100, 50, 40, 8))
    os_, ovf = pl.pallas_call(
        functools.partial(_node_body, names=nnames),
        grid=(N // BN,),
        in_specs=[pl.BlockSpec((BN, 64), lambda i: (i, 0)),
                  pl.BlockSpec((BN, 128), lambda i: (i, 0)),
                  pl.BlockSpec((BN, 128), lambda i: (i, 0)),
                  pl.BlockSpec((BN, 128), lambda i: (i, 0)),
                  pl.BlockSpec((BN, 128), lambda i: (i, 0)),
                  pl.BlockSpec((BN, 128), lambda i: (i, 0))] +
                 [_full_spec(w) for w in nvals],
        out_specs=[pl.BlockSpec((BN, 64), lambda i: (i, 0)),
                   pl.BlockSpec((BN, 128), lambda i: (i, 0))],
        out_shape=[jax.ShapeDtypeStruct((N, 64), F32),
                   jax.ShapeDtypeStruct((N, 128), F32)],
    )(s, vflat, agg_s[0], agg_s[1], agg_v[0], agg_v[1], *nvals)

    ov = ovf.reshape(N, 8, 16).transpose(0, 2, 1)
    return os_, ov


# speedup vs baseline: 1.0094x; 1.0094x over previous
import functools

import jax
import jax.numpy as jnp
import numpy as np
from jax import lax
from jax.experimental import pallas as pl
from jax.experimental.pallas import tpu as pltpu
from jax.experimental.pallas import tpu_sc as plsc

F32 = jnp.float32


def _cayley_np():
    blades = [(), (1,), (2,), (3,), (1, 2), (1, 3), (2, 3), (1, 2, 3)]
    index = {b: i for i, b in enumerate(blades)}
    C = np.zeros((8, 8, 8), dtype=np.float32)
    for i, a in enumerate(blades):
        for j, b in enumerate(blades):
            lst = list(a) + list(b)
            sign = 1.0
            changed = True
            while changed:
                changed = False
                for k in range(len(lst) - 1):
                    if lst[k] > lst[k + 1]:
                        lst[k], lst[k + 1] = lst[k + 1], lst[k]
                        sign = -sign
                        changed = True
            out = []
            k = 0
            while k < len(lst):
                if k + 1 < len(lst) and lst[k] == lst[k + 1]:
                    k += 2
                else:
                    out.append(lst[k])
                    k += 1
            C[i, j, index[tuple(out)]] += sign
    return C


_C = _cayley_np()
_I16 = np.eye(16, dtype=np.float32)
_TT = np.kron(np.ones((1, 8), np.float32), _I16)
_RS16 = np.kron(np.ones((8, 1), np.float32), _I16)
_RS32 = np.kron(np.ones((8, 1), np.float32), np.eye(32, dtype=np.float32))
_PERM = np.stack([np.kron(_C[:, j, :], _I16) for j in range(8)])


def _kron8(Wt):
    return jnp.kron(jnp.eye(8, dtype=F32), Wt)


def _b0(b, h):
    return jnp.concatenate([b, jnp.zeros((7 * h,), F32)]).reshape(1, -1)


def _row(x):
    return x.reshape(1, -1)


def _prep(lin, gp, ln, has_dpos):
    W = {}
    dlT, drT, whT = lin['dl_W'].T, lin['dr_W'].T, lin['wh_W'].T
    W['KlA'], W['KlB'] = _kron8(dlT[:16]), _kron8(dlT[16:])
    W['bl'] = _b0(lin['dl_b'], 32)
    W['KrA'], W['KrB'] = _kron8(drT[:16]), _kron8(drT[16:])
    W['br'] = _b0(lin['dr_b'], 32)
    W['KhA'], W['KhB'] = _kron8(whT[:16]), _kron8(whT[16:])
    W['bh'] = _b0(lin['wh_b'], 32)
    W['doT'] = lin['do_W'].T
    wsT = lin['ws_W'].T
    W['wsA'], W['wsB'] = wsT[0:64], wsT[64:128]
    if has_dpos:
        W['wsD'] = wsT[128:129]
        W['wsV'] = wsT[129:]
    else:
        W['wsV'] = wsT[128:]
    W['ws_b'] = _row(lin['ws_b'])
    W['Kwv'] = _kron8(lin['wv_W'].T)
    W['bwv'] = _b0(lin['wv_b'], 16)
    W['wsvT'] = lin['wsv_W'].T
    W['bwsv'] = _row(lin['wsv_b'])
    Kl = _kron8(gp['gp_l_W'].T)
    blv = jnp.concatenate([gp['gp_l_b'], jnp.zeros((112,), F32)])
    P = jnp.asarray(_PERM)
    W['L'] = jnp.einsum('ab,jbc->jac', Kl, P)
    W['bL'] = jnp.einsum('b,jbc->jc', blv, P)
    W['gKr'] = _kron8(gp['gp_r_W'].T)
    W['gbr'] = _b0(gp['gp_r_b'], 16)
    W['Ko'] = _kron8(gp['gp_o_W'].T)
    W['bo'] = _b0(gp['gp_o_b'], 16)
    W['a1'] = _row(jnp.tile(gp['gp_ln_a'], 8))
    W['Kw2'] = _kron8(gp['wv_W'].T)
    W['bw2'] = _b0(gp['wv_b'], 16)
    s2T = gp['s2v_W'].T
    W['S1'], W['S2'] = s2T[:16], s2T[16:]
    W['s2b'] = _row(gp['s2v_b'])
    W['g'] = _row(ln['ln_g'])
    W['b'] = _row(ln['ln_b'])
    W['a2'] = _row(jnp.tile(ln['mv_a'], 8))
    W['RS16'] = jnp.asarray(_RS16)
    W['RS32'] = jnp.asarray(_RS32)
    W['TT'] = jnp.asarray(_TT)
    return W


def _cvp_chain(sA, sB, dpos, vA, vB, W):
    rs16, rs32, tt = W['RS16'], W['RS32'], W['TT']
    vl = vA @ W['KlA'] + vB @ W['KlB'] + W['bl']
    vr = vA @ W['KrA'] + vB @ W['KrB'] + W['br']
    dot = (vl * vr) @ rs32
    vn = dot @ W['doT']
    s_out = sA @ W['wsA'] + sB @ W['wsB'] + vn @ W['wsV'] + W['ws_b']
    if dpos is not None:
        s_out = s_out + dpos * W['wsD']
    vh = vA @ W['KhA'] + vB @ W['KhB'] + W['bh']
    v_out = vh @ W['Kwv'] + W['bwv']
    gate = jax.nn.sigmoid(s_out) @ W['wsvT'] + W['bwsv']
    v_out = v_out * (jax.nn.sigmoid(gate) @ tt)
    ms = jnp.maximum(s_out, 0.0)
    mv = v_out
    vr2 = mv @ W['gKr'] + W['gbr']
    vo = (mv @ W['L'][0] + W['bL'][0]) * (vr2[:, 0:16] @ tt)
    for j in range(1, 8):
        vo = vo + (mv @ W['L'][j] + W['bL'][j]) * (vr2[:, 16 * j:16 * (j + 1)] @ tt)
    vo = vo @ W['Ko'] + W['bo']
    nrm = jnp.sqrt((vo * vo) @ rs16)
    vo = W['a1'] * vo / (jnp.mean(nrm, axis=1, keepdims=True) + 1e-6)
    v2 = vo + mv
    sc0 = v2[:, 0:16]
    g = jax.nn.sigmoid(np.float32(np.sqrt(2.0 / np.pi)) *
                       (2.0 * (sc0 + 0.044715 * sc0 ** 3)))
    v2 = v2 * (g @ tt)
    v2 = v2 @ W['Kw2'] + W['bw2']
    scal = v2[:, 0:16] @ W['S1'] + ms @ W['S2'] + W['s2b']
    v2 = jnp.concatenate([scal, v2[:, 16:]], axis=1)
    mu = jnp.mean(ms, axis=1, keepdims=True)
    var = jnp.mean((ms - mu) ** 2, axis=1, keepdims=True)
    ms = (ms - mu) * lax.rsqrt(var + 1e-5) * W['g'] + W['b']
    nb = jnp.sqrt((v2 * v2) @ rs16)
    v2 = W['a2'] * v2 / (jnp.mean(nb, axis=1, keepdims=True) + 1e-6)
    return ms, v2


def _edge_body(rec_ref, send_ref, *rest, names, halves=1):
    outs_ref, outv_ref = rest[-2], rest[-1]
    W = dict(zip(names, [r[...] for r in rest[:-2]]))
    hb = rec_ref.shape[0] // halves
    for h in range(halves):
        sl = slice(h * hb, (h + 1) * hb)
        R = rec_ref[sl, :]
        S = send_ref[sl, :]
        dpos = S[:, 64:65] - R[:, 64:65]
        ms, mv = _cvp_chain(R[:, 0:64], S[:, 0:64], dpos,
                            R[:, 128:256], S[:, 128:256], W)
        outs_ref[sl, :] = jnp.concatenate([ms, jnp.zeros_like(ms)], axis=1)
        outv_ref[sl, :] = mv


def _node_body(s_ref, v_ref, as0_ref, as1_ref, av0_ref, av1_ref, *rest, names):
    os_ref, ov_ref = rest[-2], rest[-1]
    W = dict(zip(names, [r[...] for r in rest[:-2]]))
    aggs = as0_ref[...] + as1_ref[...]
    aggv = av0_ref[...] + av1_ref[...]
    ms, mv = _cvp_chain(s_ref[...], aggs[:, 0:64], None,
                        v_ref[...], aggv, W)
    os_ref[...] = ms
    ov_ref[...] = mv


def _pick_block(n, prefs):
    for b in prefs:
        if n % b == 0:
            return b
    return 1


def _full_spec(w):
    nd = w.ndim
    return pl.BlockSpec(w.shape, lambda i, _nd=nd: (0,) * _nd)


def _gather_edges(table, rec, send):
    N, D = table.shape
    E = rec.shape[0]
    NW = 32
    epw = E // NW
    CH = _pick_block(epw, (400, 320, 200, 160, 80, 40, 8))
    iters = epw // CH
    mesh = plsc.VectorSubcoreMesh(core_axis_name="c", subcore_axis_name="s")

    @functools.partial(
        pl.kernel, mesh=mesh,
        out_type=[jax.ShapeDtypeStruct((E, D), F32),
                  jax.ShapeDtypeStruct((E, D), F32)],
        scratch_types=[pltpu.VMEM((CH,), jnp.int32),
                       pltpu.VMEM((CH, D), F32),
                       pltpu.SemaphoreType.DMA],
    )
    def k(table_h, rec_h, send_h, rec_o, send_o, idx_v, rows_v, sem):
        wid = lax.axis_index("s") * 2 + lax.axis_index("c")
        base = wid * epw

        def body(t, carry):
            off = base + t * CH
            pltpu.sync_copy(rec_h.at[pl.ds(off, CH)], idx_v)
            pltpu.async_copy(table_h.at[idx_v], rows_v, sem).wait()
            pltpu.sync_copy(rows_v, rec_o.at[pl.ds(off, CH)])
            pltpu.sync_copy(send_h.at[pl.ds(off, CH)], idx_v)
            pltpu.async_copy(table_h.at[idx_v], rows_v, sem).wait()
            pltpu.sync_copy(rows_v, send_o.at[pl.ds(off, CH)])
            return carry

        lax.fori_loop(0, iters, body, 0)

    return k(table, rec, send)


def _scatter_msgs(msg, rec, zeros_nd):
    E, Dm = msg.shape
    N = zeros_nd.shape[0]
    NW = 32
    epw = E // NW
    CH = _pick_block(epw, (200, 160, 80, 40, 8))
    iters = epw // CH
    RC = _pick_block(N, (200, 400, 80, 40, 16, 8))
    n_chunks = N // RC
    per_tile = -(-n_chunks // 16)
    mesh = plsc.VectorSubcoreMesh(core_axis_name="c", subcore_axis_name="s")

    @functools.partial(
        pl.kernel, mesh=mesh,
        out_type=jax.ShapeDtypeStruct((2, N, Dm), F32),
        scratch_types=[pltpu.VMEM((CH,), jnp.int32),
                       pltpu.VMEM((CH, Dm), F32),
                       pltpu.VMEM_SHARED((N, Dm), F32)],
    )
    def k(msg_h, rec_h, zero_h, out_h, idx_v, rows_v, acc):
        c = lax.axis_index("c")
        s = lax.axis_index("s")
        wid = s * 2 + c

        def rows_body(t, carry, copy_out):
            cid = s * per_tile + t

            @pl.when(cid < n_chunks)
            def _():
                off = cid * RC
                if copy_out:
                    pltpu.sync_copy(acc.at[pl.ds(off, RC)],
                                    out_h.at[c, pl.ds(off, RC)])
                else:
                    pltpu.sync_copy(zero_h.at[pl.ds(off, RC)],
                                    acc.at[pl.ds(off, RC)])
            return carry

        lax.fori_loop(0, per_tile, functools.partial(rows_body, copy_out=False), 0)
        plsc.subcore_barrier()

        def body(t, carry):
            off = wid * epw + t * CH
            pltpu.sync_copy(rec_h.at[pl.ds(off, CH)], idx_v)
            pltpu.sync_copy(msg_h.at[pl.ds(off, CH)], rows_v)
            pltpu.sync_copy(rows_v, acc.at[idx_v], add=True)
            return carry

        lax.fori_loop(0, iters, body, 0)
        plsc.subcore_barrier()
        lax.fori_loop(0, per_tile, functools.partial(rows_body, copy_out=True), 0)

    return k(msg, rec, zeros_nd)


def kernel(s, v, pos, edge_index, params):
    N = s.shape[0]
    E = edge_index.shape[1]
    vflat = v.transpose(0, 2, 1).reshape(N, 128)
    table = jnp.concatenate(
        [s, pos[:, None], jnp.zeros((N, 63), F32), vflat], axis=1)
    send = edge_index[0]
    rec = edge_index[1]

    rec_feat, send_feat = _gather_edges(table, rec, send)

    We = _prep(params['edge_lin'], params['edge_gp'], params['edge_ln'], True)
    enames = tuple(We.keys())
    evals = [We[k] for k in enames]
    BE = _pick_block(E, (512, 640, 400, 320, 256, 128, 64, 32, 8))
    msg_s, msg_v = pl.pallas_call(
        functools.partial(_edge_body, names=enames, halves=2),
        grid=(E // BE,),
        in_specs=[pl.BlockSpec((BE, 256), lambda i: (i, 0)),
                  pl.BlockSpec((BE, 256), lambda i: (i, 0))] +
                 [_full_spec(w) for w in evals],
        out_specs=[pl.BlockSpec((BE, 128), lambda i: (i, 0)),
                   pl.BlockSpec((BE, 128), lambda i: (i, 0))],
        out_shape=[jax.ShapeDtypeStruct((E, 128), F32),
                   jax.ShapeDtypeStruct((E, 128), F32)],
    )(rec_feat, send_feat, *evals)

    zeros_nd = jnp.zeros((N, 128), F32)
    agg_s = _scatter_msgs(msg_s, rec, zeros_nd)
    agg_v = _scatter_msgs(msg_v, rec, zeros_nd)

    Wn = _prep(params['node_lin'], params['node_gp'], params['node_ln'], False)
    nnames = tuple(Wn.keys())
    nvals = [Wn[k] for k in nnames]
    BN = _pick_block(N, (1000, 500, 250, 200, 100, 50, 40, 8))
    os_, ovf = pl.pallas_call(
        functools.partial(_node_body, names=nnames),
        grid=(N // BN,),
        in_specs=[pl.BlockSpec((BN, 64), lambda i: (i, 0)),
                  pl.BlockSpec((BN, 128), lambda i: (i, 0)),
                  pl.BlockSpec((BN, 128), lambda i: (i, 0)),
                  pl.BlockSpec((BN, 128), lambda i: (i, 0)),
                  pl.BlockSpec((BN, 128), lambda i: (i, 0)),
                  pl.BlockSpec((BN, 128), lambda i: (i, 0))] +
                 [_full_spec(w) for w in nvals],
        out_specs=[pl.BlockSpec((BN, 64), lambda i: (i, 0)),
                   pl.BlockSpec((BN, 128), lambda i: (i, 0))],
        out_shape=[jax.ShapeDtypeStruct((N, 64), F32),
                   jax.ShapeDtypeStruct((N, 128), F32)],
    )(s, vflat, agg_s[0], agg_s[1], agg_v[0], agg_v[1], *nvals)

    ov = ovf.reshape(N, 8, 16).transpose(0, 2, 1)
    return os_, ov


# speedup vs baseline: 1.7477x; 1.7315x over previous
import functools

import jax
import jax.numpy as jnp
import numpy as np
from jax import lax
from jax.experimental import pallas as pl
from jax.experimental.pallas import tpu as pltpu
from jax.experimental.pallas import tpu_sc as plsc

F32 = jnp.float32


def _cayley_np():
    blades = [(), (1,), (2,), (3,), (1, 2), (1, 3), (2, 3), (1, 2, 3)]
    index = {b: i for i, b in enumerate(blades)}
    C = np.zeros((8, 8, 8), dtype=np.float32)
    for i, a in enumerate(blades):
        for j, b in enumerate(blades):
            lst = list(a) + list(b)
            sign = 1.0
            changed = True
            while changed:
                changed = False
                for k in range(len(lst) - 1):
                    if lst[k] > lst[k + 1]:
                        lst[k], lst[k + 1] = lst[k + 1], lst[k]
                        sign = -sign
                        changed = True
            out = []
            k = 0
            while k < len(lst):
                if k + 1 < len(lst) and lst[k] == lst[k + 1]:
                    k += 2
                else:
                    out.append(lst[k])
                    k += 1
            C[i, j, index[tuple(out)]] += sign
    return C


_C = _cayley_np()
_I16 = np.eye(16, dtype=np.float32)
_TT = np.kron(np.ones((1, 8), np.float32), _I16)
_RS16 = np.kron(np.ones((8, 1), np.float32), _I16)
_RS32 = np.kron(np.ones((8, 1), np.float32), np.eye(32, dtype=np.float32))
_PERM = np.stack([np.kron(_C[:, j, :], _I16) for j in range(8)])


def _kron8(Wt):
    return jnp.kron(jnp.eye(8, dtype=F32), Wt)


def _b0(b, h):
    return jnp.concatenate([b, jnp.zeros((7 * h,), F32)]).reshape(1, -1)


def _row(x):
    return x.reshape(1, -1)


def _prep(lin, gp, ln, has_dpos):
    W = {}
    dlT, drT, whT = lin['dl_W'].T, lin['dr_W'].T, lin['wh_W'].T
    W['KlA'], W['KlB'] = _kron8(dlT[:16]), _kron8(dlT[16:])
    W['bl'] = _b0(lin['dl_b'], 32)
    W['KrA'], W['KrB'] = _kron8(drT[:16]), _kron8(drT[16:])
    W['br'] = _b0(lin['dr_b'], 32)
    W['KhA'], W['KhB'] = _kron8(whT[:16]), _kron8(whT[16:])
    W['bh'] = _b0(lin['wh_b'], 32)
    W['doT'] = lin['do_W'].T
    wsT = lin['ws_W'].T
    W['wsA'], W['wsB'] = wsT[0:64], wsT[64:128]
    if has_dpos:
        W['wsD'] = wsT[128:129]
        W['wsV'] = wsT[129:]
    else:
        W['wsV'] = wsT[128:]
    W['ws_b'] = _row(lin['ws_b'])
    W['Kwv'] = _kron8(lin['wv_W'].T)
    W['bwv'] = _b0(lin['wv_b'], 16)
    W['wsvT'] = lin['wsv_W'].T
    W['bwsv'] = _row(lin['wsv_b'])
    Kl = _kron8(gp['gp_l_W'].T)
    blv = jnp.concatenate([gp['gp_l_b'], jnp.zeros((112,), F32)])
    P = jnp.asarray(_PERM)
    W['L'] = jnp.einsum('ab,jbc->jac', Kl, P)
    W['bL'] = jnp.einsum('b,jbc->jc', blv, P)
    W['gKr'] = _kron8(gp['gp_r_W'].T)
    W['gbr'] = _b0(gp['gp_r_b'], 16)
    W['Ko'] = _kron8(gp['gp_o_W'].T)
    W['bo'] = _b0(gp['gp_o_b'], 16)
    W['a1'] = _row(jnp.tile(gp['gp_ln_a'], 8))
    W['Kw2'] = _kron8(gp['wv_W'].T)
    W['bw2'] = _b0(gp['wv_b'], 16)
    s2T = gp['s2v_W'].T
    W['S1'], W['S2'] = s2T[:16], s2T[16:]
    W['s2b'] = _row(gp['s2v_b'])
    W['g'] = _row(ln['ln_g'])
    W['b'] = _row(ln['ln_b'])
    W['a2'] = _row(jnp.tile(ln['mv_a'], 8))
    W['RS16'] = jnp.asarray(_RS16)
    W['RS32'] = jnp.asarray(_RS32)
    W['TT'] = jnp.asarray(_TT)
    return W


def _cvp_chain(sA, sB, dpos, vA, vB, W):
    rs16, rs32, tt = W['RS16'], W['RS32'], W['TT']
    vl = vA @ W['KlA'] + vB @ W['KlB'] + W['bl']
    vr = vA @ W['KrA'] + vB @ W['KrB'] + W['br']
    dot = (vl * vr) @ rs32
    vn = dot @ W['doT']
    s_out = sA @ W['wsA'] + sB @ W['wsB'] + vn @ W['wsV'] + W['ws_b']
    if dpos is not None:
        s_out = s_out + dpos * W['wsD']
    vh = vA @ W['KhA'] + vB @ W['KhB'] + W['bh']
    v_out = vh @ W['Kwv'] + W['bwv']
    gate = jax.nn.sigmoid(s_out) @ W['wsvT'] + W['bwsv']
    v_out = v_out * (jax.nn.sigmoid(gate) @ tt)
    ms = jnp.maximum(s_out, 0.0)
    mv = v_out
    vr2 = mv @ W['gKr'] + W['gbr']
    vo = (mv @ W['L'][0] + W['bL'][0]) * (vr2[:, 0:16] @ tt)
    for j in range(1, 8):
        vo = vo + (mv @ W['L'][j] + W['bL'][j]) * (vr2[:, 16 * j:16 * (j + 1)] @ tt)
    vo = vo @ W['Ko'] + W['bo']
    nrm = jnp.sqrt((vo * vo) @ rs16)
    vo = W['a1'] * vo / (jnp.mean(nrm, axis=1, keepdims=True) + 1e-6)
    v2 = vo + mv
    sc0 = v2[:, 0:16]
    g = jax.nn.sigmoid(np.float32(np.sqrt(2.0 / np.pi)) *
                       (2.0 * (sc0 + 0.044715 * sc0 ** 3)))
    v2 = v2 * (g @ tt)
    v2 = v2 @ W['Kw2'] + W['bw2']
    scal = v2[:, 0:16] @ W['S1'] + ms @ W['S2'] + W['s2b']
    v2 = jnp.concatenate([scal, v2[:, 16:]], axis=1)
    mu = jnp.mean(ms, axis=1, keepdims=True)
    var = jnp.mean((ms - mu) ** 2, axis=1, keepdims=True)
    ms = (ms - mu) * lax.rsqrt(var + 1e-5) * W['g'] + W['b']
    nb = jnp.sqrt((v2 * v2) @ rs16)
    v2 = W['a2'] * v2 / (jnp.mean(nb, axis=1, keepdims=True) + 1e-6)
    return ms, v2


def _edge_body(rec_ref, send_ref, *rest, names, halves=1):
    outs_ref, outv_ref = rest[-2], rest[-1]
    W = dict(zip(names, [r[...] for r in rest[:-2]]))
    hb = rec_ref.shape[0] // halves
    for h in range(halves):
        sl = slice(h * hb, (h + 1) * hb)
        R = rec_ref[sl, :]
        S = send_ref[sl, :]
        dpos = S[:, 64:65] - R[:, 64:65]
        ms, mv = _cvp_chain(R[:, 0:64], S[:, 0:64], dpos,
                            R[:, 128:256], S[:, 128:256], W)
        outs_ref[sl, :] = jnp.concatenate([ms, jnp.zeros_like(ms)], axis=1)
        outv_ref[sl, :] = mv


def _node_body(s_ref, v_ref, as0_ref, as1_ref, av0_ref, av1_ref, *rest, names):
    os_ref, ov_ref = rest[-2], rest[-1]
    W = dict(zip(names, [r[...] for r in rest[:-2]]))
    aggs = as0_ref[...] + as1_ref[...]
    aggv = av0_ref[...] + av1_ref[...]
    ms, mv = _cvp_chain(s_ref[...], aggs[:, 0:64], None,
                        v_ref[...], aggv, W)
    os_ref[...] = ms
    ov_ref[...] = mv


def _pick_block(n, prefs):
    for b in prefs:
        if n % b == 0:
            return b
    return 1


def _full_spec(w):
    nd = w.ndim
    return pl.BlockSpec(w.shape, lambda i, _nd=nd: (0,) * _nd)


def _gather_edges(table, rec, send):
    N, D = table.shape
    E = rec.shape[0]
    NW = 32
    epw = E // NW
    CH = _pick_block(epw, (400, 320, 200, 160, 80, 40, 8))
    iters = epw // CH
    mesh = plsc.VectorSubcoreMesh(core_axis_name="c", subcore_axis_name="s")

    @functools.partial(
        pl.kernel, mesh=mesh,
        out_type=[jax.ShapeDtypeStruct((E, D), F32),
                  jax.ShapeDtypeStruct((E, D), F32)],
        scratch_types=[pltpu.VMEM((CH,), jnp.int32),
                       pltpu.VMEM((CH, D), F32),
                       pltpu.SemaphoreType.DMA],
    )
    def k(table_h, rec_h, send_h, rec_o, send_o, idx_v, rows_v, sem):
        wid = lax.axis_index("s") * 2 + lax.axis_index("c")
        base = wid * epw

        def body(t, carry):
            off = base + t * CH
            pltpu.sync_copy(rec_h.at[pl.ds(off, CH)], idx_v)
            pltpu.async_copy(table_h.at[idx_v], rows_v, sem).wait()
            pltpu.sync_copy(rows_v, rec_o.at[pl.ds(off, CH)])
            pltpu.sync_copy(send_h.at[pl.ds(off, CH)], idx_v)
            pltpu.async_copy(table_h.at[idx_v], rows_v, sem).wait()
            pltpu.sync_copy(rows_v, send_o.at[pl.ds(off, CH)])
            return carry

        lax.fori_loop(0, iters, body, 0)

    return k(table, rec, send)


def _scatter_kinds(msg_s, msg_v, rec, zeros_nd):
    E, Dm = msg_s.shape
    N = zeros_nd.shape[0]
    ept = E // 16
    CH = _pick_block(ept, (200, 160, 80, 40, 8))
    iters = ept // CH
    RC = _pick_block(N, (200, 400, 80, 40, 16, 8))
    n_chunks = N // RC
    per_tile = -(-n_chunks // 16)
    mesh = plsc.VectorSubcoreMesh(core_axis_name="c", subcore_axis_name="s")

    @functools.partial(
        pl.kernel, mesh=mesh,
        out_type=jax.ShapeDtypeStruct((2, N, Dm), F32),
        scratch_types=[pltpu.VMEM((CH,), jnp.int32),
                       pltpu.VMEM((CH, Dm), F32),
                       pltpu.VMEM_SHARED((N, Dm), F32)],
    )
    def k(msgs_h, msgv_h, rec_h, zero_h, out_h, idx_v, rows_v, acc):
        c = lax.axis_index("c")
        s = lax.axis_index("s")

        def rows_body(t, carry, copy_out):
            cid = s * per_tile + t

            @pl.when(cid < n_chunks)
            def _():
                off = cid * RC
                if copy_out:
                    pltpu.sync_copy(acc.at[pl.ds(off, RC)],
                                    out_h.at[c, pl.ds(off, RC)])
                else:
                    pltpu.sync_copy(zero_h.at[pl.ds(off, RC)],
                                    acc.at[pl.ds(off, RC)])
            return carry

        lax.fori_loop(0, per_tile, functools.partial(rows_body, copy_out=False), 0)
        plsc.subcore_barrier()

        def body(t, carry, msg_h):
            off = s * ept + t * CH
            pltpu.sync_copy(rec_h.at[pl.ds(off, CH)], idx_v)
            pltpu.sync_copy(msg_h.at[pl.ds(off, CH)], rows_v)
            pltpu.sync_copy(rows_v, acc.at[idx_v], add=True)
            return carry

        @pl.when(c == 0)
        def _():
            lax.fori_loop(0, iters, functools.partial(body, msg_h=msgs_h), 0)

        @pl.when(c == 1)
        def _():
            lax.fori_loop(0, iters, functools.partial(body, msg_h=msgv_h), 0)

        plsc.subcore_barrier()
        lax.fori_loop(0, per_tile, functools.partial(rows_body, copy_out=True), 0)

    return k(msg_s, msg_v, rec, zeros_nd)


def kernel(s, v, pos, edge_index, params):
    N = s.shape[0]
    E = edge_index.shape[1]
    vflat = v.transpose(0, 2, 1).reshape(N, 128)
    table = jnp.concatenate(
        [s, pos[:, None], jnp.zeros((N, 63), F32), vflat], axis=1)
    send = edge_index[0]
    rec = edge_index[1]

    We = _prep(params['edge_lin'], params['edge_gp'], params['edge_ln'], True)
    enames = tuple(We.keys())
    evals = [We[k] for k in enames]

    def edge_stage(rec_feat, send_feat, Ec):
        BE = _pick_block(Ec, (640, 512, 400, 320, 256, 128, 64, 32, 8))
        return pl.pallas_call(
            functools.partial(_edge_body, names=enames),
            grid=(Ec // BE,),
            in_specs=[pl.BlockSpec((BE, 256), lambda i: (i, 0)),
                      pl.BlockSpec((BE, 256), lambda i: (i, 0))] +
                     [_full_spec(w) for w in evals],
            out_specs=[pl.BlockSpec((BE, 128), lambda i: (i, 0)),
                       pl.BlockSpec((BE, 128), lambda i: (i, 0))],
            out_shape=[jax.ShapeDtypeStruct((Ec, 128), F32),
                       jax.ShapeDtypeStruct((Ec, 128), F32)],
        )(rec_feat, send_feat, *evals)

    zeros_nd = jnp.zeros((N, 128), F32)
    E2 = E // 2
    g0 = _gather_edges(table, rec[:E2], send[:E2])
    g1 = _gather_edges(table, rec[E2:], send[E2:])
    m0s, m0v = edge_stage(g0[0], g0[1], E2)
    m1s, m1v = edge_stage(g1[0], g1[1], E2)
    sc0 = _scatter_kinds(m0s, m0v, rec[:E2], zeros_nd)
    sc1 = _scatter_kinds(m1s, m1v, rec[E2:], zeros_nd)

    Wn = _prep(params['node_lin'], params['node_gp'], params['node_ln'], False)
    nnames = tuple(Wn.keys())
    nvals = [Wn[k] for k in nnames]
    BN = _pick_block(N, (1000, 500, 250, 200, 100, 50, 40, 8))
    os_, ovf = pl.pallas_call(
        functools.partial(_node_body, names=nnames),
        grid=(N // BN,),
        in_specs=[pl.BlockSpec((BN, 64), lambda i: (i, 0)),
                  pl.BlockSpec((BN, 128), lambda i: (i, 0)),
                  pl.BlockSpec((BN, 128), lambda i: (i, 0)),
                  pl.BlockSpec((BN, 128), lambda i: (i, 0)),
                  pl.BlockSpec((BN, 128), lambda i: (i, 0)),
                  pl.BlockSpec((BN, 128), lambda i: (i, 0))] +
                 [_full_spec(w) for w in nvals],
        out_specs=[pl.BlockSpec((BN, 64), lambda i: (i, 0)),
                   pl.BlockSpec((BN, 128), lambda i: (i, 0))],
        out_shape=[jax.ShapeDtypeStruct((N, 64), F32),
                   jax.ShapeDtypeStruct((N, 128), F32)],
    )(s, vflat, sc0[0], sc1[0], sc0[1], sc1[1], *nvals)

    ov = ovf.reshape(N, 8, 16).transpose(0, 2, 1)
    return os_, ov
